# Initial kernel scaffold; baseline (speedup 1.0000x reference)
import jax
import jax.numpy as jnp
from jax.experimental import pallas as pl


def kernel(values, coord):
    raise NotImplementedError("write your pallas kernel here")


# speedup vs baseline: 68.1699x; 68.1699x over previous
import dataclasses

import jax
import jax.numpy as jnp
from jax import lax
from jax.experimental import pallas as pl
from jax.experimental.pallas import tpu as pltpu
from jax.experimental.pallas import tpu_sc as plsc

SIZE = 256
B = 64
N = 16384
NUM_PIX = SIZE * SIZE
NC = 2
NS = 16
NW = NC * NS
BPW = B // NW
L = 16


def _splat_body(c0_hbm, c1_hbm, val_hbm, out_hbm, acc_v, c0_v, c1_v, val_v):
    wid = lax.axis_index("s") * NC + lax.axis_index("c")
    zeros = jnp.zeros((L,), jnp.float32)
    onesf = jnp.ones((L,), jnp.float32)
    ones_i = jnp.ones((L,), jnp.int32)
    zeros_i = jnp.zeros((L,), jnp.int32)

    for r in range(BPW):
        b = wid * BPW + r

        @pl.loop(0, NUM_PIX, step=L)
        def _(i):
            acc_v[pl.ds(i, L)] = zeros

        pltpu.sync_copy(c0_hbm.at[b], c0_v)
        pltpu.sync_copy(c1_hbm.at[b], c1_v)
        pltpu.sync_copy(val_hbm.at[b], val_v)

        @pl.loop(0, N, step=L)
        def _(i):
            c0 = c0_v[pl.ds(i, L)]
            c1 = c1_v[pl.ds(i, L)]
            v = val_v[pl.ds(i, L)]
            f0 = c0.astype(jnp.int32)
            f1 = c1.astype(jnp.int32)
            fr0 = c0 - f0.astype(jnp.float32)
            fr1 = c1 - f1.astype(jnp.float32)
            int0 = fr0 == zeros
            int1 = fr1 == zeros
            d0c = jnp.where(int0, onesf, onesf - fr0)
            d1c = jnp.where(int1, onesf, onesf - fr1)
            c0c = f0 + jnp.where(int0, zeros_i, ones_i)
            c1c = f1 + jnp.where(int1, zeros_i, ones_i)
            row_f = f0 * SIZE
            row_c = c0c * SIZE
            plsc.addupdate_scatter(acc_v, [row_f + f1], d0c * d1c * v)
            plsc.addupdate_scatter(acc_v, [row_f + c1c], d0c * fr1 * v)
            plsc.addupdate_scatter(acc_v, [row_c + f1], fr0 * d1c * v)
            plsc.addupdate_scatter(acc_v, [row_c + c1c], fr0 * fr1 * v)

        pltpu.sync_copy(acc_v, out_hbm.at[b])


def kernel(values, coord):
    c0 = coord[:, :, 0]
    c1 = coord[:, :, 1]
    mesh = plsc.VectorSubcoreMesh(core_axis_name="c", subcore_axis_name="s")
    cp = pltpu.CompilerParams()
    if "needs_layout_passes" in pltpu.CompilerParams.__dataclass_fields__:
        cp = dataclasses.replace(cp, needs_layout_passes=False)
    splat = pl.kernel(
        _splat_body,
        out_type=jax.ShapeDtypeStruct((B, NUM_PIX), jnp.float32),
        mesh=mesh,
        scratch_types=[
            pltpu.VMEM((NUM_PIX,), jnp.float32),
            pltpu.VMEM((N,), jnp.float32),
            pltpu.VMEM((N,), jnp.float32),
            pltpu.VMEM((N,), jnp.float32),
        ],
        compiler_params=cp,
    )
    img = splat(c0, c1, values)
    return img.reshape(B, 1, SIZE, SIZE)


# speedup vs baseline: 94.4898x; 1.3861x over previous
import dataclasses

import jax
import jax.numpy as jnp
from jax import lax
from jax.experimental import pallas as pl
from jax.experimental.pallas import tpu as pltpu
from jax.experimental.pallas import tpu_sc as plsc

SIZE = 256
B = 64
N = 16384
NUM_PIX = SIZE * SIZE
NC = 2
NS = 16
NW = NC * NS
BPW = B // NW
L = 16


def _splat_body(c0_hbm, c1_hbm, val_hbm, out_hbm, acc_v, c0_v, c1_v, val_v, sem):
    wid = lax.axis_index("s") * NC + lax.axis_index("c")
    zeros = jnp.zeros((L,), jnp.float32)
    onesf = jnp.ones((L,), jnp.float32)
    ones_i = jnp.ones((L,), jnp.int32)
    maxrow = jnp.full((L,), SIZE - 1, jnp.int32)

    for r in range(BPW):
        b = wid * BPW + r

        cp0 = pltpu.async_copy(c0_hbm.at[b], c0_v, sem)
        cp1 = pltpu.async_copy(c1_hbm.at[b], c1_v, sem)
        cp2 = pltpu.async_copy(val_hbm.at[b], val_v, sem)

        @pl.loop(0, NUM_PIX, step=4 * L)
        def _(i):
            acc_v[pl.ds(i, L)] = zeros
            acc_v[pl.ds(i + L, L)] = zeros
            acc_v[pl.ds(i + 2 * L, L)] = zeros
            acc_v[pl.ds(i + 3 * L, L)] = zeros

        cp0.wait()
        cp1.wait()
        cp2.wait()

        @pl.loop(0, N, step=L)
        def _(i):
            c0 = c0_v[pl.ds(i, L)]
            c1 = c1_v[pl.ds(i, L)]
            v = val_v[pl.ds(i, L)]
            f0 = c0.astype(jnp.int32)
            f1 = c1.astype(jnp.int32)
            fr0 = c0 - f0.astype(jnp.float32)
            fr1 = c1 - f1.astype(jnp.float32)
            d0c = onesf - fr0
            d1c = onesf - fr1
            c0c = jnp.minimum(f0 + ones_i, maxrow)
            c1c = jnp.minimum(f1 + ones_i, maxrow)
            row_f = f0 * SIZE
            row_c = c0c * SIZE
            av = d0c * v
            bv = fr0 * v
            plsc.addupdate_scatter(acc_v, [row_f + f1], av * d1c)
            plsc.addupdate_scatter(acc_v, [row_f + c1c], av * fr1)
            plsc.addupdate_scatter(acc_v, [row_c + f1], bv * d1c)
            plsc.addupdate_scatter(acc_v, [row_c + c1c], bv * fr1)

        pltpu.sync_copy(acc_v, out_hbm.at[b])


def kernel(values, coord):
    c0 = coord[:, :, 0]
    c1 = coord[:, :, 1]
    mesh = plsc.VectorSubcoreMesh(core_axis_name="c", subcore_axis_name="s")
    cp = pltpu.CompilerParams()
    if "needs_layout_passes" in pltpu.CompilerParams.__dataclass_fields__:
        cp = dataclasses.replace(cp, needs_layout_passes=False)
    splat = pl.kernel(
        _splat_body,
        out_type=jax.ShapeDtypeStruct((B, NUM_PIX), jnp.float32),
        mesh=mesh,
        scratch_types=[
            pltpu.VMEM((NUM_PIX,), jnp.float32),
            pltpu.VMEM((N,), jnp.float32),
            pltpu.VMEM((N,), jnp.float32),
            pltpu.VMEM((N,), jnp.float32),
            pltpu.SemaphoreType.DMA,
        ],
        compiler_params=cp,
    )
    img = splat(c0, c1, values)
    return img.reshape(B, 1, SIZE, SIZE)
